# Initial kernel scaffold; baseline (speedup 1.0000x reference)
import jax
import jax.numpy as jnp
from jax.experimental import pallas as pl


def kernel(user_emb, item_emb, edge_vals, edge_index, users, items):
    raise NotImplementedError("write your pallas kernel here")


# speedup vs baseline: 2.3808x; 2.3808x over previous
import functools

import jax
import jax.numpy as jnp
from jax import lax
from jax.experimental import pallas as pl
from jax.experimental.pallas import tpu as pltpu
from jax.experimental.pallas import tpu_sc as plsc

N_USERS = 25000
N_ITEMS = 25000
N_TOTAL = N_USERS + N_ITEMS
EMB = 64
N_EDGES = 800000
BATCH = 4096

NC = 2
NS = 16
L = 16

HALF = N_TOTAL // NC
ROWS_PER_TILE = 1568
ACC_ROWS = NS * ROWS_PER_TILE
TRASH = 25040
LAST_ROWS = HALF - (NS - 1) * ROWS_PER_TILE

C = 80
EPT = N_EDGES // NS
NCHUNK = EPT // C

P = BATCH // (NC * NS)

def _lane_broadcast(v16, j):
    idx = jnp.full((L, 1), j, jnp.int32)
    return lax.gather(
        v16, idx,
        lax.GatherDimensionNumbers(
            offset_dims=(), collapsed_slice_dims=(0,), start_index_map=(0,)),
        slice_sizes=(1,),
        mode=lax.GatherScatterMode.PROMISE_IN_BOUNDS,
    )


_mesh = plsc.VectorSubcoreMesh(
    core_axis_name="c", subcore_axis_name="s", num_cores=NC, num_subcores=NS
)


@functools.partial(
    pl.kernel,
    out_type=jax.ShapeDtypeStruct((N_TOTAL, EMB), jnp.float32),
    mesh=_mesh,
    compiler_params=pltpu.CompilerParams(use_tc_tiling_on_sc=False, needs_layout_passes=False),
    scratch_types=[
        pltpu.VMEM((C,), jnp.int32),
        pltpu.VMEM((C,), jnp.int32),
        pltpu.VMEM((C,), jnp.float32),
        pltpu.VMEM((C, EMB), jnp.float32),
        pltpu.VMEM_SHARED((ACC_ROWS, EMB), jnp.float32),
        pltpu.SemaphoreType.DMA,
    ],
)
def _layer(emb_hbm, src_hbm, dst_hbm, val_hbm, zeros_hbm, out_hbm,
           sidx, didx, vvals, rows, acc, sem):
    c = lax.axis_index("c")
    s = lax.axis_index("s")
    pltpu.sync_copy(zeros_hbm, acc.at[pl.ds(s * ROWS_PER_TILE, ROWS_PER_TILE)])
    plsc.subcore_barrier()

    base = c * HALF
    edge0 = s * EPT

    def chunk_body(i, carry):
        off = edge0 + i * C
        pltpu.sync_copy(src_hbm.at[pl.ds(off, C)], sidx)
        pltpu.async_copy(emb_hbm.at[sidx], rows, sem).wait()
        pltpu.sync_copy(dst_hbm.at[pl.ds(off, C)], didx)
        pltpu.sync_copy(val_hbm.at[pl.ds(off, C)], vvals)

        for g in range(C // L):
            v16 = vvals[pl.ds(g * L, L)]
            for j in range(L):
                vb = _lane_broadcast(v16, j)
                e = g * L + j
                for q in range(EMB // L):
                    sl = pl.ds(q * L, L)
                    rows[e, sl] = rows[e, sl] * vb

        for g in range(C // L):
            sl = pl.ds(g * L, L)
            d = didx[sl] - base
            ok = (d >= 0) & (d < HALF)
            didx[sl] = jnp.where(ok, d, TRASH)

        pltpu.sync_copy(rows, acc.at[didx], add=True)
        return carry

    lax.fori_loop(0, NCHUNK, chunk_body, 0)
    plsc.subcore_barrier()

    row0 = s * ROWS_PER_TILE
    out0 = c * HALF + row0

    @pl.when(s < NS - 1)
    def _():
        pltpu.sync_copy(acc.at[pl.ds(row0, ROWS_PER_TILE)],
                        out_hbm.at[pl.ds(out0, ROWS_PER_TILE)])

    @pl.when(s == NS - 1)
    def _():
        pltpu.sync_copy(acc.at[pl.ds(row0, LAST_ROWS)],
                        out_hbm.at[pl.ds(out0, LAST_ROWS)])


@functools.partial(
    pl.kernel,
    out_type=jax.ShapeDtypeStruct((BATCH,), jnp.float32),
    mesh=_mesh,
    compiler_params=pltpu.CompilerParams(use_tc_tiling_on_sc=False, needs_layout_passes=False),
    scratch_types=[
        pltpu.VMEM((P,), jnp.int32),
        pltpu.VMEM((P,), jnp.int32),
        pltpu.VMEM((P, EMB), jnp.float32),
        pltpu.VMEM((P, EMB), jnp.float32),
        pltpu.VMEM((P, EMB), jnp.float32),
        pltpu.VMEM((P,), jnp.float32),
        pltpu.SemaphoreType.DMA,
    ],
)
def _score(e0, e1, e2, e3, users_hbm, items_hbm, gamma_hbm,
           uid, iid, tmp, usum, isum, gbuf, sem):
    c = lax.axis_index("c")
    s = lax.axis_index("s")
    wid = s * NC + c
    p0 = wid * P
    pltpu.sync_copy(users_hbm.at[pl.ds(p0, P)], uid)
    pltpu.sync_copy(items_hbm.at[pl.ds(p0, P)], iid)
    for g in range(P // L):
        sl = pl.ds(g * L, L)
        iid[sl] = iid[sl] + N_USERS

    pltpu.async_copy(e0.at[uid], usum, sem).wait()
    pltpu.async_copy(e0.at[iid], isum, sem).wait()

    def accum(dst_ref, tab, idx_ref):
        pltpu.async_copy(tab.at[idx_ref], tmp, sem).wait()

        def add_row(p, carry):
            for q in range(EMB // L):
                sl = pl.ds(q * L, L)
                dst_ref[p, sl] = dst_ref[p, sl] + tmp[p, sl]
            return carry

        lax.fori_loop(0, P, add_row, 0)

    for tab in (e1, e2, e3):
        accum(usum, tab, uid)
        accum(isum, tab, iid)

    lanes = jax.lax.broadcasted_iota(jnp.int32, (L,), 0)
    for g in range(P // L):
        acc = jnp.zeros((L,), jnp.float32)
        for j in range(L):
            p = g * L + j
            t = usum[p, pl.ds(0, L)] * isum[p, pl.ds(0, L)]
            for q in range(1, EMB // L):
                sl = pl.ds(q * L, L)
                t = t + usum[p, sl] * isum[p, sl]
            acc = jnp.where(lanes == j, jnp.sum(t), acc)
        gbuf[pl.ds(g * L, L)] = acc * jnp.float32(0.0625)

    pltpu.sync_copy(gbuf, gamma_hbm.at[pl.ds(p0, P)])


def kernel(user_emb, item_emb, edge_vals, edge_index, users, items):
    e0 = jnp.concatenate([user_emb, item_emb], axis=0)
    src = edge_index[0]
    dst = edge_index[1]
    zeros = jnp.zeros((ROWS_PER_TILE, EMB), jnp.float32)
    e1 = _layer(e0, src, dst, edge_vals, zeros)
    e2 = _layer(e1, src, dst, edge_vals, zeros)
    e3 = _layer(e2, src, dst, edge_vals, zeros)
    return _score(e0, e1, e2, e3, users, items)


# speedup vs baseline: 2.6803x; 1.1258x over previous
import functools

import jax
import jax.numpy as jnp
from jax import lax
from jax.experimental import pallas as pl
from jax.experimental.pallas import tpu as pltpu
from jax.experimental.pallas import tpu_sc as plsc

N_USERS = 25000
N_ITEMS = 25000
N_TOTAL = N_USERS + N_ITEMS
EMB = 64
HEMB = EMB // 2
N_BOTH = 2 * N_TOTAL
N_EDGES = 800000
BATCH = 4096

NC = 2
NS = 16
L = 16

RPT = N_TOTAL // NS
C = 80
EPT = N_EDGES // NS
NCHUNK = EPT // C

P = BATCH // (NC * NS)


def _lane_broadcast(v16, j):
    idx = jnp.full((L, 1), j, jnp.int32)
    return lax.gather(
        v16, idx,
        lax.GatherDimensionNumbers(
            offset_dims=(), collapsed_slice_dims=(0,), start_index_map=(0,)),
        slice_sizes=(1,),
        mode=lax.GatherScatterMode.PROMISE_IN_BOUNDS,
    )


_mesh = plsc.VectorSubcoreMesh(
    core_axis_name="c", subcore_axis_name="s", num_cores=NC, num_subcores=NS
)

_params = pltpu.CompilerParams(use_tc_tiling_on_sc=False,
                               needs_layout_passes=False)


@functools.partial(
    pl.kernel,
    out_type=jax.ShapeDtypeStruct((N_BOTH, HEMB), jnp.float32),
    mesh=_mesh,
    compiler_params=_params,
    scratch_types=[
        pltpu.VMEM((C,), jnp.int32),
        pltpu.VMEM((C,), jnp.int32),
        pltpu.VMEM((C,), jnp.float32),
        pltpu.VMEM((C, HEMB), jnp.float32),
        pltpu.VMEM_SHARED((N_TOTAL, HEMB), jnp.float32),
        pltpu.SemaphoreType.DMA,
    ],
)
def _layer(emb_hbm, src_hbm, dst_hbm, val_hbm, zeros_hbm, out_hbm,
           sidx, didx, vvals, rows, acc, sem):
    c = lax.axis_index("c")
    s = lax.axis_index("s")
    pltpu.sync_copy(zeros_hbm, acc.at[pl.ds(s * RPT, RPT)])
    plsc.subcore_barrier()

    rowoff = c * N_TOTAL
    edge0 = s * EPT

    def chunk_body(i, carry):
        off = edge0 + i * C
        pltpu.sync_copy(src_hbm.at[pl.ds(off, C)], sidx)
        for g in range(C // L):
            sl = pl.ds(g * L, L)
            sidx[sl] = sidx[sl] + rowoff
        pltpu.async_copy(emb_hbm.at[sidx], rows, sem).wait()
        pltpu.sync_copy(dst_hbm.at[pl.ds(off, C)], didx)
        pltpu.sync_copy(val_hbm.at[pl.ds(off, C)], vvals)

        for g in range(C // L):
            v16 = vvals[pl.ds(g * L, L)]
            for j in range(L):
                vb = _lane_broadcast(v16, j)
                e = g * L + j
                for q in range(HEMB // L):
                    sl = pl.ds(q * L, L)
                    rows[e, sl] = rows[e, sl] * vb

        pltpu.sync_copy(rows, acc.at[didx], add=True)
        return carry

    lax.fori_loop(0, NCHUNK, chunk_body, 0)
    plsc.subcore_barrier()

    row0 = s * RPT
    pltpu.sync_copy(acc.at[pl.ds(row0, RPT)],
                    out_hbm.at[pl.ds(c * N_TOTAL + row0, RPT)])


@functools.partial(
    pl.kernel,
    out_type=jax.ShapeDtypeStruct((BATCH,), jnp.float32),
    mesh=_mesh,
    compiler_params=_params,
    scratch_types=[
        pltpu.VMEM((P,), jnp.int32),
        pltpu.VMEM((P,), jnp.int32),
        pltpu.VMEM((P,), jnp.int32),
        pltpu.VMEM((P,), jnp.int32),
        pltpu.VMEM((P, HEMB), jnp.float32),
        pltpu.VMEM((P, HEMB), jnp.float32),
        pltpu.VMEM((P, HEMB), jnp.float32),
        pltpu.VMEM((P, HEMB), jnp.float32),
        pltpu.VMEM((P, HEMB), jnp.float32),
        pltpu.VMEM((P,), jnp.float32),
        pltpu.SemaphoreType.DMA,
    ],
)
def _score(e0, e1, e2, e3, users_hbm, items_hbm, gamma_hbm,
           ulo, uhi, ilo, ihi, tmp, uslo, ushi, islo, ishi, gbuf, sem):
    c = lax.axis_index("c")
    s = lax.axis_index("s")
    wid = s * NC + c
    p0 = wid * P
    pltpu.sync_copy(users_hbm.at[pl.ds(p0, P)], ulo)
    pltpu.sync_copy(items_hbm.at[pl.ds(p0, P)], ilo)
    for g in range(P // L):
        sl = pl.ds(g * L, L)
        uhi[sl] = ulo[sl] + N_TOTAL
        ilo[sl] = ilo[sl] + N_USERS
        ihi[sl] = ilo[sl] + N_TOTAL

    def accum(dst_ref, tab, idx_ref, first):
        if first:
            pltpu.async_copy(tab.at[idx_ref], dst_ref, sem).wait()
            return
        pltpu.async_copy(tab.at[idx_ref], tmp, sem).wait()

        def add_row(p, carry):
            for q in range(HEMB // L):
                sl = pl.ds(q * L, L)
                dst_ref[p, sl] = dst_ref[p, sl] + tmp[p, sl]
            return carry

        lax.fori_loop(0, P, add_row, 0)

    for t, tab in enumerate((e0, e1, e2, e3)):
        accum(uslo, tab, ulo, t == 0)
        accum(ushi, tab, uhi, t == 0)
        accum(islo, tab, ilo, t == 0)
        accum(ishi, tab, ihi, t == 0)

    lanes = jax.lax.broadcasted_iota(jnp.int32, (L,), 0)
    for g in range(P // L):
        acc = jnp.zeros((L,), jnp.float32)
        for j in range(L):
            p = g * L + j
            t = uslo[p, pl.ds(0, L)] * islo[p, pl.ds(0, L)]
            for q in range(1, HEMB // L):
                sl = pl.ds(q * L, L)
                t = t + uslo[p, sl] * islo[p, sl]
            for q in range(HEMB // L):
                sl = pl.ds(q * L, L)
                t = t + ushi[p, sl] * ishi[p, sl]
            acc = jnp.where(lanes == j, jnp.sum(t), acc)
        gbuf[pl.ds(g * L, L)] = acc * jnp.float32(0.0625)

    pltpu.sync_copy(gbuf, gamma_hbm.at[pl.ds(p0, P)])


def kernel(user_emb, item_emb, edge_vals, edge_index, users, items):
    all_emb = jnp.concatenate([user_emb, item_emb], axis=0)
    e0 = (all_emb.reshape(N_TOTAL, 2, HEMB).transpose(1, 0, 2)
          .reshape(N_BOTH, HEMB))
    src = edge_index[0]
    dst = edge_index[1]
    zeros = jnp.zeros((RPT, HEMB), jnp.float32)
    e1 = _layer(e0, src, dst, edge_vals, zeros)
    e2 = _layer(e1, src, dst, edge_vals, zeros)
    e3 = _layer(e2, src, dst, edge_vals, zeros)
    return _score(e0, e1, e2, e3, users, items)
